# baseline (speedup 1.0000x reference)
import functools
import math

import jax
import jax.numpy as jnp
from jax import lax
from jax.experimental import pallas as pl
from jax.experimental.pallas import tpu as pltpu
from jax.experimental.pallas import tpu_sc as plsc

D_MODEL = 64
SCALE = 8.0

_info = plsc.get_sparse_core_info()
_NC = _info.num_cores
_NS = _info.num_subcores
_NW = _NC * _NS
_LANES = _info.num_lanes

CHUNK = 512


def _emb_kernel(B: int):
    b_per_w = B // _NW
    n_chunks = b_per_w // CHUNK
    mesh = plsc.VectorSubcoreMesh(core_axis_name="c", subcore_axis_name="s")

    @functools.partial(
        pl.kernel,
        mesh=mesh,
        out_type=jax.ShapeDtypeStruct((B, D_MODEL), jnp.float32),
        scratch_types=[
            pltpu.VMEM((CHUNK,), jnp.int32),
            pltpu.VMEM((CHUNK, D_MODEL), jnp.float32),
            pltpu.SemaphoreType.DMA,
        ],
        compiler_params=pltpu.CompilerParams(use_tc_tiling_on_sc=False),
    )
    def k(idx_hbm, table_hbm, out_hbm, idx_v, rows_v, sem):
        wid = lax.axis_index("s") * _NC + lax.axis_index("c")
        base = wid * b_per_w

        def chunk_body(ci, _):
            off = base + ci * CHUNK
            pltpu.sync_copy(idx_hbm.at[pl.ds(off, CHUNK)], idx_v)
            pltpu.async_copy(table_hbm.at[idx_v], rows_v, sem).wait()

            def scale_body(r, _):
                for c in range(D_MODEL // _LANES):
                    sl = pl.ds(c * _LANES, _LANES)
                    rows_v[r, sl] = rows_v[r, sl] * SCALE
                return 0

            lax.fori_loop(0, CHUNK, scale_body, 0)
            pltpu.sync_copy(rows_v, out_hbm.at[pl.ds(off, CHUNK)])
            return 0

        lax.fori_loop(0, n_chunks, chunk_body, 0)

    return k


def kernel(x, lut):
    b, s = x.shape
    B = b * s
    flat_idx = x.reshape(B).astype(jnp.int32)
    out = _emb_kernel(B)(flat_idx, lut)
    return out.reshape(b, s, D_MODEL)


# speedup vs baseline: 1.0995x; 1.0995x over previous
import functools

import jax
import jax.numpy as jnp
from jax import lax
from jax.experimental import pallas as pl
from jax.experimental.pallas import tpu as pltpu
from jax.experimental.pallas import tpu_sc as plsc

D = 64
SEQ = 200
SCALE = 8.0
LANES = 16

_info = plsc.get_sparse_core_info()
_NC = _info.num_cores
_NS = _info.num_subcores
_NW = _NC * _NS

S = 4
G = 40
GPS = SEQ // G


def _emb_kernel(B: int, V: int):
    sent_per_w = B // _NW
    n_chunks = sent_per_w // S
    n_pairs = n_chunks // 2
    mesh = plsc.VectorSubcoreMesh(core_axis_name="c", subcore_axis_name="s")

    @functools.partial(
        pl.kernel,
        mesh=mesh,
        out_type=jax.ShapeDtypeStruct((B, SEQ, D), jnp.float32),
        scratch_types=[
            pltpu.VMEM((S, SEQ), jnp.int32),
            pltpu.VMEM((S, SEQ, D), jnp.float32),
            pltpu.VMEM((S, SEQ, D), jnp.float32),
            pltpu.SemaphoreType.DMA,
            pltpu.SemaphoreType.DMA,
            pltpu.SemaphoreType.DMA,
        ],
        compiler_params=pltpu.CompilerParams(use_tc_tiling_on_sc=False),
    )
    def k(idx_hbm, table_hbm, out_hbm, idx_v, rows0, rows1, gsem, osem0, osem1):
        wid = lax.axis_index("s") * _NC + lax.axis_index("c")
        base = wid * sent_per_w
        rows = (rows0, rows1)
        osem = (osem0, osem1)

        def chunk_step(c, b):
            s0 = base + c * S
            @pl.when(c >= 2)
            def _():
                pltpu.make_async_copy(
                    rows[b], out_hbm.at[pl.ds(s0 - 2 * S, S)], osem[b]
                ).wait()

            pltpu.sync_copy(idx_hbm.at[pl.ds(s0, S)], idx_v)
            copies = []
            for s in range(S):
                for h in range(GPS):
                    copies.append(pltpu.async_copy(
                        table_hbm.at[idx_v.at[s, pl.ds(h * G, G)]],
                        rows[b].at[s, pl.ds(h * G, G)],
                        gsem,
                    ))
            for cp in copies:
                cp.wait()

            for s in range(S):
                @plsc.parallel_loop(0, SEQ, unroll=4)
                def _(r):
                    for cc in range(D // LANES):
                        sl = pl.ds(cc * LANES, LANES)
                        rows[b][s, r, sl] = rows[b][s, r, sl] * SCALE

            pltpu.make_async_copy(
                rows[b], out_hbm.at[pl.ds(s0, S)], osem[b]
            ).start()

        def pair_body(p, _):
            chunk_step(2 * p, 0)
            chunk_step(2 * p + 1, 1)
            return 0

        lax.fori_loop(0, n_pairs, pair_body, 0)

        last0 = base + (n_chunks - 2) * S
        pltpu.make_async_copy(rows[0], out_hbm.at[pl.ds(last0, S)], osem[0]).wait()
        pltpu.make_async_copy(
            rows[1], out_hbm.at[pl.ds(last0 + S, S)], osem[1]
        ).wait()

    return k


def kernel(x, lut):
    B, seq = x.shape
    assert seq == SEQ and lut.shape[1] == D
    return _emb_kernel(B, lut.shape[0])(x.astype(jnp.int32), lut)
